# Initial kernel scaffold; baseline (speedup 1.0000x reference)
import jax
import jax.numpy as jnp
from jax.experimental import pallas as pl


def kernel(x, edge_type, edge_index, edge_label, node_table, edge_table, eps, We, be, W1, b1, W2, b2, Wh1, bh1, Wh2, bh2):
    raise NotImplementedError("write your pallas kernel here")


# speedup vs baseline: 4.1110x; 4.1110x over previous
import functools
import jax
import jax.numpy as jnp
from jax import lax
from jax.experimental import pallas as pl
from jax.experimental.pallas import tpu as pltpu
from jax.experimental.pallas import tpu_sc as plsc

NC = 2
NS = 16
NW = NC * NS
CH = 128


def _embed_body(x_ref, tab_ref, o_ref):
    xi = x_ref[...]
    z = jnp.broadcast_to(tab_ref[0:1, :], o_ref.shape)
    for k in range(1, 4):
        z = jnp.where(xi == k, tab_ref[k:k + 1, :], z)
    o_ref[...] = z


def _embed(x, node_table, N, H):
    Bn = 1000
    return pl.pallas_call(
        _embed_body,
        grid=(N // Bn,),
        in_specs=[
            pl.BlockSpec((Bn, 1), lambda i: (i, 0)),
            pl.BlockSpec((4, H), lambda i: (0, 0)),
        ],
        out_specs=pl.BlockSpec((Bn, H), lambda i: (i, 0)),
        out_shape=jax.ShapeDtypeStruct((N, H), jnp.float32),
    )(x, node_table)


def _z4_body(z_ref, et_ref, w_ref, b_ref, o_ref):
    e = pl.program_id(0)
    row = et_ref[pl.ds(e, 1), :]
    t = jnp.dot(row, w_ref[...],
                preferred_element_type=jnp.float32) + b_ref[...]
    o_ref[0] = jnp.maximum(z_ref[...] + t, 0.0)


def _z4(z, edge_table, We_l, be_l, N, H):
    Bn = 1000
    return pl.pallas_call(
        _z4_body,
        grid=(4, N // Bn),
        in_specs=[
            pl.BlockSpec((Bn, H), lambda e, i: (i, 0)),
            pl.BlockSpec((4, H), lambda e, i: (0, 0)),
            pl.BlockSpec((H, H), lambda e, i: (0, 0)),
            pl.BlockSpec((1, H), lambda e, i: (0, 0)),
        ],
        out_specs=pl.BlockSpec((1, Bn, H), lambda e, i: (e, i, 0)),
        out_shape=jax.ShapeDtypeStruct((4, N, H), jnp.float32),
    )(z, edge_table, We_l, be_l)


def _update_body(z_ref, a_ref, s_ref, w1_ref, b1_ref, w2_ref, b2_ref, o_ref):
    a = a_ref[0] + a_ref[1]
    h = z_ref[...] * s_ref[0, 0] + a
    h = jnp.maximum(jnp.dot(h, w1_ref[...],
                            preferred_element_type=jnp.float32) + b1_ref[...], 0.0)
    h = jnp.dot(h, w2_ref[...], preferred_element_type=jnp.float32) + b2_ref[...]
    o_ref[...] = jnp.maximum(h, 0.0)


def _update(z, aggr2, scale, W1_l, b1_l, W2_l, b2_l, rows, Bn, H):
    return pl.pallas_call(
        _update_body,
        grid=(rows // Bn,),
        in_specs=[
            pl.BlockSpec((Bn, H), lambda i: (i, 0)),
            pl.BlockSpec((2, Bn, H), lambda i: (0, i, 0)),
            pl.BlockSpec((1, 1), lambda i: (0, 0)),
            pl.BlockSpec((H, H), lambda i: (0, 0)),
            pl.BlockSpec((1, H), lambda i: (0, 0)),
            pl.BlockSpec((H, H), lambda i: (0, 0)),
            pl.BlockSpec((1, H), lambda i: (0, 0)),
        ],
        out_specs=pl.BlockSpec((Bn, H), lambda i: (i, 0)),
        out_shape=jax.ShapeDtypeStruct((rows, H), jnp.float32),
    )(z, aggr2, scale, W1_l, b1_l, W2_l, b2_l)


def _head_body(z_ref, w1_ref, b1_ref, w2t_ref, b2_ref, o_ref):
    B = o_ref.shape[0]
    g = jnp.concatenate([z_ref[:B], z_ref[B:]], axis=1)
    hh = jnp.maximum(jnp.dot(g, w1_ref[...],
                             preferred_element_type=jnp.float32) + b1_ref[...], 0.0)
    pred = jnp.sum(hh * w2t_ref[...], axis=1, keepdims=True) + b2_ref[...]
    o_ref[...] = pred


def _head(z2b, Wh1, bh1, Wh2t, bh2, B, H):
    return pl.pallas_call(
        _head_body,
        out_shape=jax.ShapeDtypeStruct((B, 1), jnp.float32),
    )(z2b, Wh1, bh1, Wh2t, bh2)


def _make_sc_aggregate(N, H, NPAD, CPW):
    RPS = NPAD // NS

    mesh = plsc.VectorSubcoreMesh(core_axis_name="c", subcore_axis_name="s",
                                  num_cores=NC, num_subcores=NS)

    @functools.partial(
        pl.kernel,
        out_type=jax.ShapeDtypeStruct((NC, NPAD, H), jnp.float32),
        mesh=mesh,
        scratch_types=[
            pltpu.VMEM((CH,), jnp.int32),
            pltpu.VMEM((CH,), jnp.int32),
            pltpu.VMEM((CH, H), jnp.float32),
            pltpu.VMEM_SHARED((NPAD, H), jnp.float32),
            pltpu.SemaphoreType.DMA,
        ],
    )
    def sc_aggr(z4_hbm, gidx_hbm, dst_hbm, zeros_hbm, out_hbm,
                idx_g, idx_s, rows, aggr, sem):
        c = lax.axis_index("c")
        s = lax.axis_index("s")
        wid = c * NS + s
        r0 = s * RPS

        pltpu.sync_copy(zeros_hbm.at[pl.ds(r0, RPS)], aggr.at[pl.ds(r0, RPS)])
        plsc.subcore_barrier()

        row_base = wid * CPW

        def body(j, carry):
            pltpu.sync_copy(gidx_hbm.at[row_base + j], idx_g)
            pltpu.async_copy(z4_hbm.at[idx_g], rows, sem).wait()
            pltpu.sync_copy(dst_hbm.at[row_base + j], idx_s)
            pltpu.sync_copy(rows, aggr.at[idx_s], add=True)
            return carry

        lax.fori_loop(0, CPW, body, 0)
        plsc.subcore_barrier()

        pltpu.sync_copy(aggr.at[pl.ds(r0, RPS)], out_hbm.at[c].at[pl.ds(r0, RPS)])

    return sc_aggr


def kernel(x, edge_type, edge_index, edge_label, node_table, edge_table, eps,
           We, be, W1, b1, W2, b2, Wh1, bh1, Wh2, bh2):
    N, H = x.shape[0], node_table.shape[1]
    E = edge_type.shape[0]
    B = edge_label.shape[0]
    L = We.shape[0]

    NPAD = ((N + NS * 8 - 1) // (NS * 8)) * (NS * 8)
    E_pad = ((E + NW * CH - 1) // (NW * CH)) * (NW * CH)
    CPW = E_pad // (NW * CH)

    src = edge_index[0]
    dst = edge_index[1]
    gidx = (edge_type * N + src).astype(jnp.int32)
    gidx = jnp.pad(gidx, (0, E_pad - E)).reshape(NW * CPW, CH)
    dstp = jnp.pad(dst, (0, E_pad - E), constant_values=N).astype(jnp.int32)
    dstp = dstp.reshape(NW * CPW, CH)
    zeros = jnp.zeros((NPAD, H), jnp.float32)

    sc_aggr = _make_sc_aggregate(N, H, NPAD, CPW)

    z = _embed(x, node_table, N, H)
    for l in range(L):
        z4 = _z4(z, edge_table, We[l], be[l].reshape(1, H), N, H)
        aggr2 = sc_aggr(z4.reshape(4 * N, H), gidx, dstp, zeros)
        scale = (1.0 + eps[l]).reshape(1, 1)
        rows = N if l < L - 1 else 2 * B
        Bn = 1000 if l < L - 1 else 2 * B
        z = _update(z, aggr2, scale, W1[l], b1[l].reshape(1, H),
                    W2[l], b2[l].reshape(1, H), rows, Bn, H)

    pred = _head(z, Wh1, bh1.reshape(1, H), Wh2.reshape(1, H),
                 bh2.reshape(1, 1), B, H)
    return (pred, edge_label)
